# baseline (speedup 1.0000x reference)
import functools

import jax
import jax.numpy as jnp
from jax.experimental import pallas as pl
from jax.experimental.pallas import tpu as pltpu


def _roi_body(fm_ref, cd_ref, sc_ref, out_ref, *, H, W):
    N = sc_ref.shape[2]
    c_blk = fm_ref.shape[1]
    cd = cd_ref[0]
    x0 = cd[0:1, :]
    x1 = cd[1:2, :]
    y0 = cd[2:3, :]
    y1 = cd[3:4, :]

    xi = jax.lax.broadcasted_iota(jnp.int32, (W, N), 0)
    colt = jnp.where((xi >= x0) & (xi < x1), 1.0, 0.0).astype(jnp.bfloat16)

    fm = fm_ref[0]
    acc = jnp.zeros((c_blk, N), jnp.float32)
    for yc in range(0, H, 16):
        rows = min(16, H - yc)
        xc = fm[:, yc:yc + rows, :].reshape(c_blk * rows, W)
        uc = jnp.dot(xc, colt, preferred_element_type=jnp.float32)
        uc = uc.reshape(c_blk, rows, N)
        yi = jax.lax.broadcasted_iota(jnp.int32, (rows, N), 0) + yc
        rc = jnp.where((yi >= y0) & (yi < y1), 1.0, 0.0).astype(jnp.float32)
        acc = acc + jnp.sum(uc * rc[None, :, :], axis=1)
    out_ref[0] = acc * sc_ref[0]


def kernel(feature_map, keypoints, mask, original_H, original_W):
    B, C, H, W = feature_map.shape
    N = keypoints.shape[1]
    sx = W / original_W
    sy = H / original_H
    x, y, w, h = (keypoints[..., 0], keypoints[..., 1],
                  keypoints[..., 2], keypoints[..., 3])
    xr = jnp.clip((x * sx).astype(jnp.int32), 0, W - 1)
    yr = jnp.clip((y * sy).astype(jnp.int32), 0, H - 1)
    wr = jnp.minimum(jnp.maximum((w * sx).astype(jnp.int32), 1), W - xr)
    hr = jnp.minimum(jnp.maximum((h * sy).astype(jnp.int32), 1), H - yr)
    coords = jnp.stack([xr, xr + wr, yr, yr + hr], axis=1)
    area = (hr * wr).astype(jnp.float32)
    scale = jnp.where(mask > 0, 1.0 / area, 0.0).reshape(B, 1, N)

    fm = feature_map.astype(jnp.bfloat16)
    c_blk = 128
    grid = (B, C // c_blk)
    out = pl.pallas_call(
        functools.partial(_roi_body, H=H, W=W),
        grid=grid,
        in_specs=[
            pl.BlockSpec((1, c_blk, H, W), lambda b, c: (b, c, 0, 0)),
            pl.BlockSpec((1, 4, N), lambda b, c: (b, 0, 0)),
            pl.BlockSpec((1, 1, N), lambda b, c: (b, 0, 0)),
        ],
        out_specs=pl.BlockSpec((1, c_blk, N), lambda b, c: (b, c, 0)),
        out_shape=jax.ShapeDtypeStruct((B, C, N), jnp.float32),
        compiler_params=pltpu.CompilerParams(
            dimension_semantics=("parallel", "arbitrary"),
            vmem_limit_bytes=50 * 1024 * 1024,
        ),
    )(fm, coords, scale)
    return jnp.transpose(out, (0, 2, 1))
